# Initial kernel scaffold; baseline (speedup 1.0000x reference)
import jax
import jax.numpy as jnp
from jax.experimental import pallas as pl


def kernel(x, edge_index, W1, b1, W2, b2, Wfc, bfc):
    raise NotImplementedError("write your pallas kernel here")


# speedup vs baseline: 31.0317x; 31.0317x over previous
import functools

import jax
import jax.numpy as jnp
from jax import lax
from jax.experimental import pallas as pl
from jax.experimental.pallas import tpu as pltpu
from jax.experimental.pallas import tpu_sc as plsc

N = 10000
IN_CH = 128
HID = 64
E = 320000

NC, NS = 2, 16
NW = NC * NS
BATCH = 125
EDGE_ROWS = E // BATCH
ROWS_PER_WORKER = EDGE_ROWS // NW
RCHUNK = 4
CHUNKS = ROWS_PER_WORKER // RCHUNK
NPAD = 10240
SLICE = NPAD // NS

_sc_mesh = plsc.VectorSubcoreMesh(core_axis_name="c", subcore_axis_name="s")


@functools.partial(
    pl.kernel,
    out_type=jax.ShapeDtypeStruct((NC, NPAD), jnp.float32),
    mesh=_sc_mesh,
    scratch_types=[
        pltpu.VMEM((RCHUNK, BATCH), jnp.int32),
        pltpu.VMEM((BATCH,), jnp.float32),
        pltpu.VMEM_SHARED((NPAD,), jnp.float32),
    ],
)
def _deg_kernel(dst_hbm, zeros_hbm, ones_hbm, out_hbm, idx_v, ones_v, deg_sp):
    c = lax.axis_index("c")
    s = lax.axis_index("s")
    pltpu.sync_copy(zeros_hbm.at[pl.ds(0, SLICE)], deg_sp.at[pl.ds(s * SLICE, SLICE)])
    pltpu.sync_copy(ones_hbm, ones_v)
    plsc.subcore_barrier()
    row0 = (c * NS + s) * ROWS_PER_WORKER

    def body(i, carry):
        base = pl.multiple_of(row0 + i * RCHUNK, RCHUNK)
        pltpu.sync_copy(dst_hbm.at[pl.ds(base, RCHUNK)], idx_v)
        for j in range(RCHUNK):
            pltpu.sync_copy(ones_v, deg_sp.at[idx_v.at[j]], add=True)
        return carry

    lax.fori_loop(0, CHUNKS, body, 0)
    plsc.subcore_barrier()
    pltpu.sync_copy(deg_sp.at[pl.ds(s * SLICE, SLICE)],
                    out_hbm.at[c].at[pl.ds(s * SLICE, SLICE)])


@functools.partial(
    pl.kernel,
    out_type=jax.ShapeDtypeStruct((NC, NPAD, HID), jnp.float32),
    mesh=_sc_mesh,
    scratch_types=[
        pltpu.VMEM((RCHUNK, BATCH), jnp.int32),
        pltpu.VMEM((RCHUNK, BATCH), jnp.int32),
        pltpu.VMEM((RCHUNK, BATCH, HID), jnp.float32),
        pltpu.VMEM_SHARED((NPAD, HID), jnp.float32),
        pltpu.SemaphoreType.DMA,
    ],
    compiler_params=pltpu.CompilerParams(use_tc_tiling_on_sc=False),
)
def _agg_kernel(src_hbm, dst_hbm, hn_hbm, zrows_hbm, out_hbm,
                src_v, dst_v, rows_v, agg_sp, sem):
    c = lax.axis_index("c")
    s = lax.axis_index("s")
    pltpu.sync_copy(zrows_hbm, agg_sp.at[pl.ds(s * SLICE, SLICE)])
    plsc.subcore_barrier()
    row0 = (c * NS + s) * ROWS_PER_WORKER

    def body(i, carry):
        base = pl.multiple_of(row0 + i * RCHUNK, RCHUNK)
        pltpu.sync_copy(src_hbm.at[pl.ds(base, RCHUNK)], src_v)
        pltpu.sync_copy(dst_hbm.at[pl.ds(base, RCHUNK)], dst_v)
        descs = [pltpu.async_copy(hn_hbm.at[src_v.at[j]], rows_v.at[j], sem)
                 for j in range(RCHUNK)]
        for d in descs:
            d.wait()
        for j in range(RCHUNK):
            pltpu.sync_copy(rows_v.at[j], agg_sp.at[dst_v.at[j]], add=True)
        return carry

    lax.fori_loop(0, CHUNKS, body, 0)
    plsc.subcore_barrier()
    pltpu.sync_copy(agg_sp.at[pl.ds(s * SLICE, SLICE)],
                    out_hbm.at[c].at[pl.ds(s * SLICE, SLICE)])


def _prep_body(degp_ref, x_ref, w1_ref, hn_ref, dinv_ref):
    deg = degp_ref[:, 0:1] + degp_ref[:, 1:2] + 1.0
    dinv = 1.0 / jnp.sqrt(deg)
    dinv_ref[...] = dinv
    h = jnp.dot(x_ref[...], w1_ref[...], preferred_element_type=jnp.float32)
    hn_ref[...] = h * dinv


def _mid_body(aggp_ref, hn_ref, dinv_ref, b_ref, w2_ref, out_ref):
    agg = aggp_ref[0, :N, :] + aggp_ref[1, :N, :]
    dinv = dinv_ref[...]
    a = jnp.maximum(dinv * (agg + hn_ref[...]) + b_ref[...], 0.0)
    h = jnp.dot(a, w2_ref[...], preferred_element_type=jnp.float32)
    out_ref[...] = h * dinv


def _fin_body(aggp_ref, hn_ref, dinv_ref, b_ref, wfc_ref, bfc_ref, out_ref):
    agg = aggp_ref[0, :N, :] + aggp_ref[1, :N, :]
    a = jnp.maximum(dinv_ref[...] * (agg + hn_ref[...]) + b_ref[...], 0.0)
    s = a[0:625]
    for i in range(1, 16):
        s = s + a[i * 625:(i + 1) * 625]
    t = s[0:125]
    for i in range(1, 5):
        t = t + s[i * 125:(i + 1) * 125]
    g = jnp.sum(t, axis=0, keepdims=True) * (1.0 / N)
    out_ref[...] = (jnp.sum(g * wfc_ref[...], axis=1, keepdims=True)
                    + bfc_ref[...])


_prep = pl.pallas_call(
    _prep_body,
    out_shape=(jax.ShapeDtypeStruct((N, HID), jnp.float32),
               jax.ShapeDtypeStruct((N, 1), jnp.float32)),
)
_mid = pl.pallas_call(
    _mid_body,
    out_shape=jax.ShapeDtypeStruct((N, HID), jnp.float32),
)
_fin = pl.pallas_call(
    _fin_body,
    out_shape=jax.ShapeDtypeStruct((1, 1), jnp.float32),
)


def kernel(x, edge_index, W1, b1, W2, b2, Wfc, bfc):
    src = edge_index[0].astype(jnp.int32).reshape(EDGE_ROWS, BATCH)
    dst = edge_index[1].astype(jnp.int32).reshape(EDGE_ROWS, BATCH)

    zeros1 = jnp.zeros((SLICE,), jnp.float32)
    ones1 = jnp.ones((BATCH,), jnp.float32)
    zrows = jnp.zeros((SLICE, HID), jnp.float32)

    degp = _deg_kernel(dst, zeros1, ones1)
    degp_col = degp.T[:N]

    hn1, dinv = _prep(degp_col, x, W1)
    agg1 = _agg_kernel(src, dst, hn1, zrows)
    hn2 = _mid(agg1, hn1, dinv, b1.reshape(1, HID), W2)
    agg2 = _agg_kernel(src, dst, hn2, zrows)
    out = _fin(agg2, hn2, dinv, b2.reshape(1, HID), Wfc.reshape(1, HID),
               bfc.reshape(1, 1))
    return out


# speedup vs baseline: 38.7880x; 1.2499x over previous
import functools

import jax
import jax.numpy as jnp
from jax import lax
from jax.experimental import pallas as pl
from jax.experimental.pallas import tpu as pltpu
from jax.experimental.pallas import tpu_sc as plsc

N = 10000
IN_CH = 128
HID = 64
E = 320000

NC, NS = 2, 16
NW = NC * NS
BATCH = 125
EDGE_ROWS = E // BATCH
ROWS_PER_WORKER = EDGE_ROWS // NW
RCHUNK = 4
CHUNKS = ROWS_PER_WORKER // RCHUNK
NPAD = 10240
SLICE = NPAD // NS

_sc_mesh = plsc.VectorSubcoreMesh(core_axis_name="c", subcore_axis_name="s")


@functools.partial(
    pl.kernel,
    out_type=jax.ShapeDtypeStruct((NC, NPAD), jnp.float32),
    mesh=_sc_mesh,
    scratch_types=[
        pltpu.VMEM((RCHUNK, BATCH), jnp.int32),
        pltpu.VMEM((BATCH,), jnp.float32),
        pltpu.VMEM_SHARED((NPAD,), jnp.float32),
    ],
)
def _deg_kernel(dst_hbm, zeros_hbm, ones_hbm, out_hbm, idx_v, ones_v, deg_sp):
    c = lax.axis_index("c")
    s = lax.axis_index("s")
    pltpu.sync_copy(zeros_hbm.at[pl.ds(0, SLICE)], deg_sp.at[pl.ds(s * SLICE, SLICE)])
    pltpu.sync_copy(ones_hbm, ones_v)
    plsc.subcore_barrier()
    row0 = (c * NS + s) * ROWS_PER_WORKER

    def body(i, carry):
        base = pl.multiple_of(row0 + i * RCHUNK, RCHUNK)
        pltpu.sync_copy(dst_hbm.at[pl.ds(base, RCHUNK)], idx_v)
        for j in range(RCHUNK):
            pltpu.sync_copy(ones_v, deg_sp.at[idx_v.at[j]], add=True)
        return carry

    lax.fori_loop(0, CHUNKS, body, 0)
    plsc.subcore_barrier()
    pltpu.sync_copy(deg_sp.at[pl.ds(s * SLICE, SLICE)],
                    out_hbm.at[c].at[pl.ds(s * SLICE, SLICE)])


@functools.partial(
    pl.kernel,
    out_type=jax.ShapeDtypeStruct((NC, NPAD, HID), jnp.float32),
    mesh=_sc_mesh,
    scratch_types=[
        pltpu.VMEM((2, RCHUNK, BATCH), jnp.int32),
        pltpu.VMEM((2, RCHUNK, BATCH), jnp.int32),
        pltpu.VMEM((2, RCHUNK, BATCH, HID), jnp.float32),
        pltpu.VMEM_SHARED((NPAD, HID), jnp.float32),
        pltpu.SemaphoreType.DMA,
        pltpu.SemaphoreType.DMA,
    ],
    compiler_params=pltpu.CompilerParams(use_tc_tiling_on_sc=False),
)
def _agg_kernel(src_hbm, dst_hbm, hn_hbm, zrows_hbm, out_hbm,
                src_v, dst_v, rows_v, agg_sp, sem0, sem1):
    c = lax.axis_index("c")
    s = lax.axis_index("s")
    pltpu.sync_copy(zrows_hbm, agg_sp.at[pl.ds(s * SLICE, SLICE)])
    plsc.subcore_barrier()
    row0 = (c * NS + s) * ROWS_PER_WORKER
    sems = (sem0, sem1)

    def load_and_fire(k, b):
        base = row0 + k * RCHUNK
        pltpu.sync_copy(src_hbm.at[pl.ds(base, RCHUNK)], src_v.at[b])
        pltpu.sync_copy(dst_hbm.at[pl.ds(base, RCHUNK)], dst_v.at[b])
        for j in range(RCHUNK):
            pltpu.async_copy(hn_hbm.at[src_v.at[b].at[j]],
                             rows_v.at[b].at[j], sems[b])

    def drain_and_scatter(b):
        for j in range(RCHUNK):
            pltpu.make_async_copy(hn_hbm.at[src_v.at[b].at[j]],
                                  rows_v.at[b].at[j], sems[b]).wait()
        for j in range(RCHUNK):
            pltpu.sync_copy(rows_v.at[b].at[j],
                            agg_sp.at[dst_v.at[b].at[j]], add=True)

    load_and_fire(0, 0)

    def body(p, carry):
        i = p * 2
        load_and_fire(i + 1, 1)
        drain_and_scatter(0)
        load_and_fire(i + 2, 0)
        drain_and_scatter(1)
        return carry

    lax.fori_loop(0, CHUNKS // 2 - 1, body, 0)
    load_and_fire(CHUNKS - 1, 1)
    drain_and_scatter(0)
    drain_and_scatter(1)
    plsc.subcore_barrier()
    pltpu.sync_copy(agg_sp.at[pl.ds(s * SLICE, SLICE)],
                    out_hbm.at[c].at[pl.ds(s * SLICE, SLICE)])


def _mm1_body(x_ref, w1_ref, h_ref):
    h_ref[...] = jnp.dot(x_ref[...], w1_ref[...],
                         preferred_element_type=jnp.float32)


def _scale_body(degp_ref, h_ref, hn_ref, dinv_ref):
    deg = degp_ref[:, 0:1] + degp_ref[:, 1:2] + 1.0
    dinv = 1.0 / jnp.sqrt(deg)
    dinv_ref[...] = dinv
    hn_ref[...] = h_ref[...] * dinv


def _mid_body(aggp_ref, hn_ref, dinv_ref, b_ref, w2_ref, out_ref):
    agg = aggp_ref[0, :N, :] + aggp_ref[1, :N, :]
    dinv = dinv_ref[...]
    a = jnp.maximum(dinv * (agg + hn_ref[...]) + b_ref[...], 0.0)
    h = jnp.dot(a, w2_ref[...], preferred_element_type=jnp.float32)
    out_ref[...] = h * dinv


def _fin_body(aggp_ref, hn_ref, dinv_ref, b_ref, wfc_ref, bfc_ref, out_ref):
    agg = aggp_ref[0, :N, :] + aggp_ref[1, :N, :]
    a = jnp.maximum(dinv_ref[...] * (agg + hn_ref[...]) + b_ref[...], 0.0)
    s = a[0:625]
    for i in range(1, 16):
        s = s + a[i * 625:(i + 1) * 625]
    t = s[0:125]
    for i in range(1, 5):
        t = t + s[i * 125:(i + 1) * 125]
    g = jnp.sum(t, axis=0, keepdims=True) * (1.0 / N)
    out_ref[...] = (jnp.sum(g * wfc_ref[...], axis=1, keepdims=True)
                    + bfc_ref[...])


_mm1 = pl.pallas_call(
    _mm1_body,
    out_shape=jax.ShapeDtypeStruct((N, HID), jnp.float32),
)
_scale = pl.pallas_call(
    _scale_body,
    out_shape=(jax.ShapeDtypeStruct((N, HID), jnp.float32),
               jax.ShapeDtypeStruct((N, 1), jnp.float32)),
)
_mid = pl.pallas_call(
    _mid_body,
    out_shape=jax.ShapeDtypeStruct((N, HID), jnp.float32),
)
_fin = pl.pallas_call(
    _fin_body,
    out_shape=jax.ShapeDtypeStruct((1, 1), jnp.float32),
)


def kernel(x, edge_index, W1, b1, W2, b2, Wfc, bfc):
    src = edge_index[0].astype(jnp.int32).reshape(EDGE_ROWS, BATCH)
    dst = edge_index[1].astype(jnp.int32).reshape(EDGE_ROWS, BATCH)

    zeros1 = jnp.zeros((SLICE,), jnp.float32)
    ones1 = jnp.ones((BATCH,), jnp.float32)
    zrows = jnp.zeros((SLICE, HID), jnp.float32)

    h1 = _mm1(x, W1)
    degp = _deg_kernel(dst, zeros1, ones1)
    degp_col = degp.T[:N]

    hn1, dinv = _scale(degp_col, h1)
    agg1 = _agg_kernel(src, dst, hn1, zrows)
    hn2 = _mid(agg1, hn1, dinv, b1.reshape(1, HID), W2)
    agg2 = _agg_kernel(src, dst, hn2, zrows)
    out = _fin(agg2, hn2, dinv, b2.reshape(1, HID), Wfc.reshape(1, HID),
               bfc.reshape(1, 1))
    return out


# speedup vs baseline: 42.8479x; 1.1047x over previous
import functools

import jax
import jax.numpy as jnp
from jax import lax
from jax.experimental import pallas as pl
from jax.experimental.pallas import tpu as pltpu
from jax.experimental.pallas import tpu_sc as plsc

N = 10000
IN_CH = 128
HID = 64
E = 320000

NC, NS = 2, 16
NW = NC * NS
BATCH = 125
EDGE_ROWS = E // BATCH
ROWS_PER_WORKER = EDGE_ROWS // NW
RCHUNK = 4
CHUNKS = ROWS_PER_WORKER // RCHUNK
NPAD = 10240
SLICE = NPAD // NS

_sc_mesh = plsc.VectorSubcoreMesh(core_axis_name="c", subcore_axis_name="s")


@functools.partial(
    pl.kernel,
    out_type=jax.ShapeDtypeStruct((NC, NPAD), jnp.float32),
    mesh=_sc_mesh,
    scratch_types=[
        pltpu.VMEM((RCHUNK, BATCH), jnp.int32),
        pltpu.VMEM((BATCH,), jnp.float32),
        pltpu.VMEM_SHARED((NPAD,), jnp.float32),
    ],
)
def _deg_kernel(dst_hbm, zeros_hbm, ones_hbm, out_hbm, idx_v, ones_v, deg_sp):
    c = lax.axis_index("c")
    s = lax.axis_index("s")
    pltpu.sync_copy(zeros_hbm.at[pl.ds(0, SLICE)], deg_sp.at[pl.ds(s * SLICE, SLICE)])
    pltpu.sync_copy(ones_hbm, ones_v)
    plsc.subcore_barrier()
    row0 = (c * NS + s) * ROWS_PER_WORKER

    def body(i, carry):
        base = pl.multiple_of(row0 + i * RCHUNK, RCHUNK)
        pltpu.sync_copy(dst_hbm.at[pl.ds(base, RCHUNK)], idx_v)
        for j in range(RCHUNK):
            pltpu.sync_copy(ones_v, deg_sp.at[idx_v.at[j]], add=True)
        return carry

    lax.fori_loop(0, CHUNKS, body, 0)
    plsc.subcore_barrier()
    pltpu.sync_copy(deg_sp.at[pl.ds(s * SLICE, SLICE)],
                    out_hbm.at[c].at[pl.ds(s * SLICE, SLICE)])


@functools.partial(
    pl.kernel,
    out_type=jax.ShapeDtypeStruct((NC, NPAD, HID), jnp.float32),
    mesh=_sc_mesh,
    scratch_types=[
        pltpu.VMEM((2, RCHUNK, 2, BATCH), jnp.int32),
        pltpu.VMEM((2, RCHUNK * BATCH, HID), jnp.float32),
        pltpu.VMEM_SHARED((NPAD, HID), jnp.float32),
        pltpu.SemaphoreType.DMA,
        pltpu.SemaphoreType.DMA,
        pltpu.SemaphoreType.DMA,
        pltpu.SemaphoreType.DMA,
    ],
    compiler_params=pltpu.CompilerParams(use_tc_tiling_on_sc=False),
)
def _agg_kernel(esd_hbm, hn_hbm, zrows_hbm, out_hbm,
                idx_v, rows_v, agg_sp, gsem0, gsem1, ssem0, ssem1):
    c = lax.axis_index("c")
    s = lax.axis_index("s")
    pltpu.sync_copy(zrows_hbm, agg_sp.at[pl.ds(s * SLICE, SLICE)])
    plsc.subcore_barrier()
    row0 = (c * NS + s) * ROWS_PER_WORKER
    gsems = (gsem0, gsem1)
    ssems = (ssem0, ssem1)

    def load_and_fire(k, b):
        base = row0 + k * RCHUNK
        pltpu.sync_copy(esd_hbm.at[pl.ds(base, RCHUNK)], idx_v.at[b])
        for j in range(RCHUNK):
            pltpu.async_copy(hn_hbm.at[idx_v.at[b].at[j].at[0]],
                             rows_v.at[b].at[pl.ds(j * BATCH, BATCH)], gsems[b])

    def drain_gathers(b):
        pltpu.make_async_copy(hn_hbm.at[pl.ds(0, RCHUNK * BATCH)],
                              rows_v.at[b], gsems[b]).wait()

    def fire_scatters(b):
        for j in range(RCHUNK):
            pltpu.async_copy(rows_v.at[b].at[pl.ds(j * BATCH, BATCH)],
                             agg_sp.at[idx_v.at[b].at[j].at[1]], ssems[b],
                             add=True)

    def drain_scatters(b):
        pltpu.make_async_copy(hn_hbm.at[pl.ds(0, RCHUNK * BATCH)],
                              rows_v.at[b], ssems[b]).wait()

    load_and_fire(0, 0)

    def body(p, carry):
        i = p * 2
        load_and_fire(i + 1, 1)
        drain_gathers(0)
        fire_scatters(0)
        drain_scatters(0)
        load_and_fire(i + 2, 0)
        drain_gathers(1)
        fire_scatters(1)
        drain_scatters(1)
        return carry

    lax.fori_loop(0, CHUNKS // 2 - 1, body, 0)
    load_and_fire(CHUNKS - 1, 1)
    drain_gathers(0)
    fire_scatters(0)
    drain_scatters(0)
    drain_gathers(1)
    fire_scatters(1)
    drain_scatters(1)
    plsc.subcore_barrier()
    pltpu.sync_copy(agg_sp.at[pl.ds(s * SLICE, SLICE)],
                    out_hbm.at[c].at[pl.ds(s * SLICE, SLICE)])


def _mm1_body(x_ref, w1_ref, h_ref):
    h_ref[...] = jnp.dot(x_ref[...], w1_ref[...],
                         preferred_element_type=jnp.float32)


def _scale_body(degp_ref, h_ref, hn_ref, dinv_ref):
    deg = degp_ref[:, 0:1] + degp_ref[:, 1:2] + 1.0
    dinv = 1.0 / jnp.sqrt(deg)
    dinv_ref[...] = dinv
    hn_ref[...] = h_ref[...] * dinv


def _mid_body(aggp_ref, hn_ref, dinv_ref, b_ref, w2_ref, out_ref):
    agg = aggp_ref[0, :N, :] + aggp_ref[1, :N, :]
    dinv = dinv_ref[...]
    a = jnp.maximum(dinv * (agg + hn_ref[...]) + b_ref[...], 0.0)
    h = jnp.dot(a, w2_ref[...], preferred_element_type=jnp.float32)
    out_ref[...] = h * dinv


def _fin_body(aggp_ref, hn_ref, dinv_ref, b_ref, wfc_ref, bfc_ref, out_ref):
    agg = aggp_ref[0, :N, :] + aggp_ref[1, :N, :]
    a = jnp.maximum(dinv_ref[...] * (agg + hn_ref[...]) + b_ref[...], 0.0)
    s = a[0:625]
    for i in range(1, 16):
        s = s + a[i * 625:(i + 1) * 625]
    t = s[0:125]
    for i in range(1, 5):
        t = t + s[i * 125:(i + 1) * 125]
    g = jnp.sum(t, axis=0, keepdims=True) * (1.0 / N)
    out_ref[...] = (jnp.sum(g * wfc_ref[...], axis=1, keepdims=True)
                    + bfc_ref[...])


_mm1 = pl.pallas_call(
    _mm1_body,
    out_shape=jax.ShapeDtypeStruct((N, HID), jnp.float32),
)
_scale = pl.pallas_call(
    _scale_body,
    out_shape=(jax.ShapeDtypeStruct((N, HID), jnp.float32),
               jax.ShapeDtypeStruct((N, 1), jnp.float32)),
)
_mid = pl.pallas_call(
    _mid_body,
    out_shape=jax.ShapeDtypeStruct((N, HID), jnp.float32),
)
_fin = pl.pallas_call(
    _fin_body,
    out_shape=jax.ShapeDtypeStruct((1, 1), jnp.float32),
)


def kernel(x, edge_index, W1, b1, W2, b2, Wfc, bfc):
    ei32 = edge_index.astype(jnp.int32).reshape(2, EDGE_ROWS, BATCH)
    dst = ei32[1]
    esd = ei32.transpose(1, 0, 2)

    zeros1 = jnp.zeros((SLICE,), jnp.float32)
    ones1 = jnp.ones((BATCH,), jnp.float32)
    zrows = jnp.zeros((SLICE, HID), jnp.float32)

    h1 = _mm1(x, W1)
    degp = _deg_kernel(dst, zeros1, ones1)
    degp_col = degp.T[:N]

    hn1, dinv = _scale(degp_col, h1)
    agg1 = _agg_kernel(esd, hn1, zrows)
    hn2 = _mid(agg1, hn1, dinv, b1.reshape(1, HID), W2)
    agg2 = _agg_kernel(esd, hn2, zrows)
    out = _fin(agg2, hn2, dinv, b2.reshape(1, HID), Wfc.reshape(1, HID),
               bfc.reshape(1, 1))
    return out


# speedup vs baseline: 46.5797x; 1.0871x over previous
import functools

import jax
import jax.numpy as jnp
from jax import lax
from jax.experimental import pallas as pl
from jax.experimental.pallas import tpu as pltpu
from jax.experimental.pallas import tpu_sc as plsc

N = 10000
IN_CH = 128
HID = 64
E = 320000

NC, NS = 2, 16
NW = NC * NS
BATCH = 125
EDGE_ROWS = E // BATCH
ROWS_PER_WORKER = EDGE_ROWS // NW
RCHUNK = 2
CHUNKS = ROWS_PER_WORKER // RCHUNK
NPAD = 10240
SLICE = NPAD // NS

_sc_mesh = plsc.VectorSubcoreMesh(core_axis_name="c", subcore_axis_name="s")


@functools.partial(
    pl.kernel,
    out_type=jax.ShapeDtypeStruct((NC, NPAD), jnp.float32),
    mesh=_sc_mesh,
    scratch_types=[
        pltpu.VMEM((ROWS_PER_WORKER, 2, BATCH), jnp.int32),
        pltpu.VMEM((BATCH,), jnp.float32),
        pltpu.VMEM_SHARED((NPAD,), jnp.float32),
        pltpu.SemaphoreType.DMA,
    ],
    compiler_params=pltpu.CompilerParams(use_tc_tiling_on_sc=False),
)
def _deg_kernel(esd_hbm, zeros_hbm, ones_hbm, out_hbm, idx_v, ones_v, deg_sp,
                sem):
    c = lax.axis_index("c")
    s = lax.axis_index("s")
    row0 = (c * NS + s) * ROWS_PER_WORKER
    pltpu.sync_copy(esd_hbm.at[pl.ds(row0, ROWS_PER_WORKER)], idx_v)
    pltpu.sync_copy(zeros_hbm.at[pl.ds(0, SLICE)],
                    deg_sp.at[pl.ds(s * SLICE, SLICE)])
    pltpu.sync_copy(ones_hbm, ones_v)
    plsc.subcore_barrier()

    def fire(k, carry):
        pltpu.async_copy(ones_v, deg_sp.at[idx_v.at[k].at[1]], sem, add=True)
        return carry

    lax.fori_loop(0, ROWS_PER_WORKER, fire, 0)

    def drain(k, carry):
        pltpu.make_async_copy(ones_v, deg_sp.at[idx_v.at[k].at[1]], sem).wait()
        return carry

    lax.fori_loop(0, ROWS_PER_WORKER, drain, 0)
    plsc.subcore_barrier()
    pltpu.sync_copy(deg_sp.at[pl.ds(s * SLICE, SLICE)],
                    out_hbm.at[c].at[pl.ds(s * SLICE, SLICE)])


@functools.partial(
    pl.kernel,
    out_type=jax.ShapeDtypeStruct((NC, NPAD, HID), jnp.float32),
    mesh=_sc_mesh,
    scratch_types=[
        pltpu.VMEM((ROWS_PER_WORKER, 2, BATCH), jnp.int32),
        pltpu.VMEM((3, RCHUNK * BATCH, HID), jnp.float32),
        pltpu.VMEM_SHARED((NPAD, HID), jnp.float32),
        pltpu.SemaphoreType.DMA,
        pltpu.SemaphoreType.DMA,
        pltpu.SemaphoreType.DMA,
        pltpu.SemaphoreType.DMA,
        pltpu.SemaphoreType.DMA,
        pltpu.SemaphoreType.DMA,
    ],
    compiler_params=pltpu.CompilerParams(use_tc_tiling_on_sc=False),
)
def _agg_kernel(esd_hbm, hn_hbm, zrows_hbm, out_hbm,
                idx_v, rows_v, agg_sp, g0, g1, g2, s0, s1, s2):
    c = lax.axis_index("c")
    s = lax.axis_index("s")
    row0 = (c * NS + s) * ROWS_PER_WORKER
    pltpu.sync_copy(esd_hbm.at[pl.ds(row0, ROWS_PER_WORKER)], idx_v)
    pltpu.sync_copy(zrows_hbm, agg_sp.at[pl.ds(s * SLICE, SLICE)])
    plsc.subcore_barrier()
    gsems = (g0, g1, g2)
    ssems = (s0, s1, s2)

    def fire_g(k, b):
        for j in range(RCHUNK):
            pltpu.async_copy(hn_hbm.at[idx_v.at[k * RCHUNK + j].at[0]],
                             rows_v.at[b].at[pl.ds(j * BATCH, BATCH)], gsems[b])

    def drain_g(b):
        pltpu.make_async_copy(hn_hbm.at[pl.ds(0, RCHUNK * BATCH)],
                              rows_v.at[b], gsems[b]).wait()

    def fire_s(k, b):
        for j in range(RCHUNK):
            pltpu.async_copy(rows_v.at[b].at[pl.ds(j * BATCH, BATCH)],
                             agg_sp.at[idx_v.at[k * RCHUNK + j].at[1]],
                             ssems[b], add=True)

    def drain_s(b):
        pltpu.make_async_copy(hn_hbm.at[pl.ds(0, RCHUNK * BATCH)],
                              rows_v.at[b], ssems[b]).wait()

    fire_g(0, 0)
    fire_g(1, 1)
    fire_g(2, 2)
    drain_g(0)
    fire_s(0, 0)

    def step(k, b):
        drain_s((b + 2) % 3)
        fire_g(k + 2, (b + 2) % 3)
        drain_g(b)
        fire_s(k, b)

    def body(p, carry):
        k = 3 * p + 1
        step(k, 1)
        step(k + 1, 2)
        step(k + 2, 0)
        return carry

    lax.fori_loop(0, (CHUNKS - 4) // 3, body, 0)
    step(CHUNKS - 3, (CHUNKS - 3) % 3)
    drain_g((CHUNKS - 2) % 3)
    fire_s(CHUNKS - 2, (CHUNKS - 2) % 3)
    drain_g((CHUNKS - 1) % 3)
    fire_s(CHUNKS - 1, (CHUNKS - 1) % 3)
    drain_s(0)
    drain_s(1)
    drain_s(2)
    plsc.subcore_barrier()
    pltpu.sync_copy(agg_sp.at[pl.ds(s * SLICE, SLICE)],
                    out_hbm.at[c].at[pl.ds(s * SLICE, SLICE)])


def _mm1_body(x_ref, w1_ref, h_ref):
    h_ref[...] = jnp.dot(x_ref[...], w1_ref[...],
                         preferred_element_type=jnp.float32)


def _scale_body(degp_ref, h_ref, hn_ref, dinv_ref):
    deg = degp_ref[:, 0:1] + degp_ref[:, 1:2] + 1.0
    dinv = 1.0 / jnp.sqrt(deg)
    dinv_ref[...] = dinv
    hn_ref[...] = h_ref[...] * dinv


def _mid_body(aggp_ref, hn_ref, dinv_ref, b_ref, w2_ref, out_ref):
    agg = aggp_ref[0, :N, :] + aggp_ref[1, :N, :]
    dinv = dinv_ref[...]
    a = jnp.maximum(dinv * (agg + hn_ref[...]) + b_ref[...], 0.0)
    h = jnp.dot(a, w2_ref[...], preferred_element_type=jnp.float32)
    out_ref[...] = h * dinv


def _fin_body(aggp_ref, hn_ref, dinv_ref, b_ref, wfc_ref, bfc_ref, out_ref):
    agg = aggp_ref[0, :N, :] + aggp_ref[1, :N, :]
    a = jnp.maximum(dinv_ref[...] * (agg + hn_ref[...]) + b_ref[...], 0.0)
    s = a[0:625]
    for i in range(1, 16):
        s = s + a[i * 625:(i + 1) * 625]
    t = s[0:125]
    for i in range(1, 5):
        t = t + s[i * 125:(i + 1) * 125]
    g = jnp.sum(t, axis=0, keepdims=True) * (1.0 / N)
    out_ref[...] = (jnp.sum(g * wfc_ref[...], axis=1, keepdims=True)
                    + bfc_ref[...])


_mm1 = pl.pallas_call(
    _mm1_body,
    out_shape=jax.ShapeDtypeStruct((N, HID), jnp.float32),
)
_scale = pl.pallas_call(
    _scale_body,
    out_shape=(jax.ShapeDtypeStruct((N, HID), jnp.float32),
               jax.ShapeDtypeStruct((N, 1), jnp.float32)),
)
_mid = pl.pallas_call(
    _mid_body,
    out_shape=jax.ShapeDtypeStruct((N, HID), jnp.float32),
)
_fin = pl.pallas_call(
    _fin_body,
    out_shape=jax.ShapeDtypeStruct((1, 1), jnp.float32),
)


def kernel(x, edge_index, W1, b1, W2, b2, Wfc, bfc):
    ei32 = edge_index.astype(jnp.int32).reshape(2, EDGE_ROWS, BATCH)
    esd = ei32.transpose(1, 0, 2)

    zeros1 = jnp.zeros((SLICE,), jnp.float32)
    ones1 = jnp.ones((BATCH,), jnp.float32)
    zrows = jnp.zeros((SLICE, HID), jnp.float32)

    h1 = _mm1(x, W1)
    degp = _deg_kernel(esd, zeros1, ones1)
    degp_col = degp.T[:N]

    hn1, dinv = _scale(degp_col, h1)
    agg1 = _agg_kernel(esd, hn1, zrows)
    hn2 = _mid(agg1, hn1, dinv, b1.reshape(1, HID), W2)
    agg2 = _agg_kernel(esd, hn2, zrows)
    out = _fin(agg2, hn2, dinv, b2.reshape(1, HID), Wfc.reshape(1, HID),
               bfc.reshape(1, 1))
    return out


# speedup vs baseline: 46.9169x; 1.0072x over previous
import functools

import jax
import jax.numpy as jnp
from jax import lax
from jax.experimental import pallas as pl
from jax.experimental.pallas import tpu as pltpu
from jax.experimental.pallas import tpu_sc as plsc

N = 10000
IN_CH = 128
HID = 64
E = 320000

NC, NS = 2, 16
NW = NC * NS
BATCH = 125
EDGE_ROWS = E // BATCH
ROWS_PER_WORKER = EDGE_ROWS // NW
RCHUNK = 2
CHUNKS = ROWS_PER_WORKER // RCHUNK
NPAD = 10240
SLICE = NPAD // NS

_sc_mesh = plsc.VectorSubcoreMesh(core_axis_name="c", subcore_axis_name="s")


@functools.partial(
    pl.kernel,
    out_type=jax.ShapeDtypeStruct((NC, NPAD), jnp.float32),
    mesh=_sc_mesh,
    scratch_types=[
        pltpu.VMEM((ROWS_PER_WORKER, 2, BATCH), jnp.int32),
        pltpu.VMEM((BATCH,), jnp.float32),
        pltpu.VMEM_SHARED((NPAD,), jnp.float32),
        pltpu.SemaphoreType.DMA,
    ],
    compiler_params=pltpu.CompilerParams(use_tc_tiling_on_sc=False),
)
def _deg_kernel(esd_hbm, zeros_hbm, ones_hbm, out_hbm, idx_v, ones_v, deg_sp,
                sem):
    c = lax.axis_index("c")
    s = lax.axis_index("s")
    row0 = (c * NS + s) * ROWS_PER_WORKER
    pltpu.sync_copy(esd_hbm.at[pl.ds(row0, ROWS_PER_WORKER)], idx_v)
    pltpu.sync_copy(zeros_hbm.at[pl.ds(0, SLICE)],
                    deg_sp.at[pl.ds(s * SLICE, SLICE)])
    pltpu.sync_copy(ones_hbm, ones_v)
    plsc.subcore_barrier()

    def fire(k, carry):
        pltpu.async_copy(ones_v, deg_sp.at[idx_v.at[k].at[1]], sem, add=True)
        return carry

    lax.fori_loop(0, ROWS_PER_WORKER, fire, 0)

    def drain(k, carry):
        pltpu.make_async_copy(ones_v, deg_sp.at[idx_v.at[k].at[1]], sem).wait()
        return carry

    lax.fori_loop(0, ROWS_PER_WORKER, drain, 0)
    plsc.subcore_barrier()
    pltpu.sync_copy(deg_sp.at[pl.ds(s * SLICE, SLICE)],
                    out_hbm.at[c].at[pl.ds(s * SLICE, SLICE)])


@functools.partial(
    pl.kernel,
    out_type=jax.ShapeDtypeStruct((NC, NPAD, HID), jnp.float32),
    mesh=_sc_mesh,
    scratch_types=[
        pltpu.VMEM((ROWS_PER_WORKER, 2, BATCH), jnp.int32),
        pltpu.VMEM((3, RCHUNK * BATCH, HID), jnp.float32),
        pltpu.VMEM_SHARED((NPAD, HID), jnp.float32),
        pltpu.SemaphoreType.DMA,
        pltpu.SemaphoreType.DMA,
        pltpu.SemaphoreType.DMA,
        pltpu.SemaphoreType.DMA,
        pltpu.SemaphoreType.DMA,
        pltpu.SemaphoreType.DMA,
    ],
    compiler_params=pltpu.CompilerParams(use_tc_tiling_on_sc=False),
)
def _agg_kernel(esd_hbm, hn_hbm, zrows_hbm, out_hbm,
                idx_v, rows_v, agg_sp, g0, g1, g2, s0, s1, s2):
    c = lax.axis_index("c")
    s = lax.axis_index("s")
    row0 = (c * NS + s) * ROWS_PER_WORKER
    pltpu.sync_copy(esd_hbm.at[pl.ds(row0, ROWS_PER_WORKER)], idx_v)
    pltpu.sync_copy(zrows_hbm, agg_sp.at[pl.ds(s * SLICE, SLICE)])
    plsc.subcore_barrier()
    gsems = (g0, g1, g2)
    ssems = (s0, s1, s2)

    def fire_g(k, b):
        for j in range(RCHUNK):
            pltpu.async_copy(hn_hbm.at[idx_v.at[k * RCHUNK + j].at[0]],
                             rows_v.at[b].at[pl.ds(j * BATCH, BATCH)], gsems[b])

    def drain_g(b):
        pltpu.make_async_copy(hn_hbm.at[pl.ds(0, RCHUNK * BATCH)],
                              rows_v.at[b], gsems[b]).wait()

    def fire_s(k, b):
        for j in range(RCHUNK):
            pltpu.async_copy(rows_v.at[b].at[pl.ds(j * BATCH, BATCH)],
                             agg_sp.at[idx_v.at[k * RCHUNK + j].at[1]],
                             ssems[b], add=True)

    def drain_s(b):
        pltpu.make_async_copy(hn_hbm.at[pl.ds(0, RCHUNK * BATCH)],
                              rows_v.at[b], ssems[b]).wait()

    fire_g(0, 0)
    fire_g(1, 1)
    fire_g(2, 2)
    drain_g(0)
    fire_s(0, 0)

    def step(k, b):
        drain_s((b + 2) % 3)
        fire_g(k + 2, (b + 2) % 3)
        drain_g(b)
        fire_s(k, b)

    def body(p, carry):
        k = 3 * p + 1
        step(k, 1)
        step(k + 1, 2)
        step(k + 2, 0)
        return carry

    lax.fori_loop(0, (CHUNKS - 4) // 3, body, 0)
    step(CHUNKS - 3, (CHUNKS - 3) % 3)
    drain_g((CHUNKS - 2) % 3)
    fire_s(CHUNKS - 2, (CHUNKS - 2) % 3)
    drain_g((CHUNKS - 1) % 3)
    fire_s(CHUNKS - 1, (CHUNKS - 1) % 3)
    drain_s(0)
    drain_s(1)
    drain_s(2)
    plsc.subcore_barrier()
    pltpu.sync_copy(agg_sp.at[pl.ds(s * SLICE, SLICE)],
                    out_hbm.at[c].at[pl.ds(s * SLICE, SLICE)])


def _prep_body(degp_ref, x_ref, w1_ref, hn_ref, dinv_ref):
    deg = degp_ref[:, 0:1] + degp_ref[:, 1:2] + 1.0
    dinv = 1.0 / jnp.sqrt(deg)
    dinv_ref[...] = dinv
    h = jnp.dot(x_ref[...], w1_ref[...], preferred_element_type=jnp.float32)
    hn_ref[...] = h * dinv


def _mid_body(aggp_ref, hn_ref, dinv_ref, b_ref, w2_ref, out_ref):
    agg = aggp_ref[0, :N, :] + aggp_ref[1, :N, :]
    dinv = dinv_ref[...]
    a = jnp.maximum(dinv * (agg + hn_ref[...]) + b_ref[...], 0.0)
    h = jnp.dot(a, w2_ref[...], preferred_element_type=jnp.float32)
    out_ref[...] = h * dinv


def _fin_body(aggp_ref, hn_ref, dinv_ref, b_ref, wfc_ref, bfc_ref, out_ref):
    agg = aggp_ref[0, :N, :] + aggp_ref[1, :N, :]
    a = jnp.maximum(dinv_ref[...] * (agg + hn_ref[...]) + b_ref[...], 0.0)
    s = a[0:625]
    for i in range(1, 16):
        s = s + a[i * 625:(i + 1) * 625]
    t = s[0:125]
    for i in range(1, 5):
        t = t + s[i * 125:(i + 1) * 125]
    g = jnp.sum(t, axis=0, keepdims=True) * (1.0 / N)
    out_ref[...] = (jnp.sum(g * wfc_ref[...], axis=1, keepdims=True)
                    + bfc_ref[...])


_prep = pl.pallas_call(
    _prep_body,
    out_shape=(jax.ShapeDtypeStruct((N, HID), jnp.float32),
               jax.ShapeDtypeStruct((N, 1), jnp.float32)),
)
_mid = pl.pallas_call(
    _mid_body,
    out_shape=jax.ShapeDtypeStruct((N, HID), jnp.float32),
)
_fin = pl.pallas_call(
    _fin_body,
    out_shape=jax.ShapeDtypeStruct((1, 1), jnp.float32),
)


def kernel(x, edge_index, W1, b1, W2, b2, Wfc, bfc):
    ei32 = edge_index.astype(jnp.int32).reshape(2, EDGE_ROWS, BATCH)
    esd = ei32.transpose(1, 0, 2)

    zeros1 = jnp.zeros((SLICE,), jnp.float32)
    ones1 = jnp.ones((BATCH,), jnp.float32)
    zrows = jnp.zeros((SLICE, HID), jnp.float32)

    degp = _deg_kernel(esd, zeros1, ones1)
    degp_col = degp.T[:N]

    hn1, dinv = _prep(degp_col, x, W1)
    agg1 = _agg_kernel(esd, hn1, zrows)
    hn2 = _mid(agg1, hn1, dinv, b1.reshape(1, HID), W2)
    agg2 = _agg_kernel(esd, hn2, zrows)
    out = _fin(agg2, hn2, dinv, b2.reshape(1, HID), Wfc.reshape(1, HID),
               bfc.reshape(1, 1))
    return out
